# Initial kernel scaffold; baseline (speedup 1.0000x reference)
import jax
import jax.numpy as jnp
from jax.experimental import pallas as pl


def kernel(X, graph_logits, mu_logits, alpha_param, beta_param, z, dt_val):
    raise NotImplementedError("write your pallas kernel here")


# speedup vs baseline: 12.0304x; 12.0304x over previous
import functools

import jax
import jax.numpy as jnp
from jax import lax
from jax.experimental import pallas as pl
from jax.experimental.pallas import tpu as pltpu
from jax.experimental.pallas import tpu_sc as plsc

NUM_MODES = 64
N_NODES = 4
B_TOK = 32768
MU_MIN = 0.1
MU_MAX = 1.5

_info = plsc.get_sparse_core_info()
NC = _info.num_cores
NS = _info.num_subcores
L = _info.num_lanes
NW = NC * NS
TOK_W = B_TOK // NW
STEPS = TOK_W // L
ME = NUM_MODES * N_NODES * N_NODES
N_CHUNK = NUM_MODES // L


def _sigmoid(x):
    return 1.0 / (1.0 + jnp.exp(-x))


XROWS = B_TOK * N_NODES // 128
ROW_W = XROWS // NW


@functools.partial(
    pl.kernel,
    out_type=jax.ShapeDtypeStruct((XROWS, 128), jnp.float32),
    mesh=plsc.VectorSubcoreMesh(core_axis_name="c", subcore_axis_name="s"),
    compiler_params=pltpu.CompilerParams(needs_layout_passes=False),
    scratch_types=[
        pltpu.VMEM((ROW_W, 128), jnp.float32),
        pltpu.VMEM((TOK_W,), jnp.int32),
        pltpu.VMEM((ME,), jnp.float32),
        pltpu.VMEM((ME,), jnp.float32),
        pltpu.VMEM((NUM_MODES * N_NODES,), jnp.float32),
        pltpu.VMEM((NUM_MODES,), jnp.float32),
        pltpu.VMEM((NUM_MODES,), jnp.float32),
        pltpu.VMEM((L,), jnp.float32),
        pltpu.VMEM((ROW_W, 128), jnp.float32),
        pltpu.SemaphoreType.DMA,
    ],
)
def _sc_forward(Xf, zf, Gf, muf, alf, bef, dtf, outf,
                Xv, zv, Av, Gv, muv, alv, bev, dtv, outv, sem):
    w = lax.axis_index("s") * NC + lax.axis_index("c")

    copies = [
        pltpu.async_copy(Xf.at[pl.ds(w * ROW_W, ROW_W), :], Xv, sem),
        pltpu.async_copy(zf.at[pl.ds(w * TOK_W, TOK_W)], zv, sem),
        pltpu.async_copy(Gf, Gv, sem),
        pltpu.async_copy(muf, muv, sem),
        pltpu.async_copy(alf, alv, sem),
        pltpu.async_copy(bef, bev, sem),
        pltpu.async_copy(dtf, dtv, sem),
    ]
    for c in copies:
        c.wait()

    k = lax.iota(jnp.int32, L)
    dt = dtv[...]

    OFFD = [(0, 1), (0, 2), (0, 3), (1, 2), (1, 3), (2, 3)]
    for c in range(N_CHUNK):
        mb = 256 * c + 16 * k
        g = [plsc.load_gather(Gv, [mb + p]) for p in range(16)]
        a = plsc.load_gather(alv, [16 * c + k])
        bb = plsc.load_gather(bev, [16 * c + k])
        dtb = dt * bb
        sig = {}
        for (i, j) in OFFD:
            sig[(i, j)] = _sigmoid(0.5 * (g[4 * i + j] + g[4 * j + i]))
        for (i, j) in OFFD:
            plsc.store_scatter(Av, [mb + (4 * i + j)], dtb * sig[(i, j)])
        for i in range(N_NODES):
            deg = None
            for j in range(N_NODES):
                if j == i:
                    continue
                t = sig[(min(i, j), max(i, j))]
                deg = t if deg is None else deg + t
            mus = MU_MIN + (MU_MAX - MU_MIN) * _sigmoid(
                plsc.load_gather(muv, [64 * c + 4 * k + i]))
            plsc.store_scatter(Av, [mb + 5 * i], 1.0 + dt * (mus + a * deg))

    @plsc.parallel_loop(0, STEPS, 1, unroll=2)
    def step(s):
        t16 = s * L
        zs = zv[pl.ds(t16, L)]
        xb = (t16 + k) * N_NODES
        xr = lax.shift_right_logical(xb, 7)
        xc = lax.bitwise_and(xb, 127)
        x0 = plsc.load_gather(Xv, [xr, xc])
        x1 = plsc.load_gather(Xv, [xr, xc + 1])
        x2 = plsc.load_gather(Xv, [xr, xc + 2])
        x3 = plsc.load_gather(Xv, [xr, xc + 3])
        ab = zs * 16
        d0 = plsc.load_gather(Av, [ab])
        d1 = plsc.load_gather(Av, [ab + 5])
        d2 = plsc.load_gather(Av, [ab + 10])
        d3 = plsc.load_gather(Av, [ab + 15])
        a01 = plsc.load_gather(Av, [ab + 1])
        a02 = plsc.load_gather(Av, [ab + 2])
        a03 = plsc.load_gather(Av, [ab + 3])
        a12 = plsc.load_gather(Av, [ab + 6])
        a13 = plsc.load_gather(Av, [ab + 7])
        a23 = plsc.load_gather(Av, [ab + 11])
        o0 = d0 * x0 + a01 * x1 + a02 * x2 + a03 * x3 - dt * (x0 * x0 * x0)
        o1 = a01 * x0 + d1 * x1 + a12 * x2 + a13 * x3 - dt * (x1 * x1 * x1)
        o2 = a02 * x0 + a12 * x1 + d2 * x2 + a23 * x3 - dt * (x2 * x2 * x2)
        o3 = a03 * x0 + a13 * x1 + a23 * x2 + d3 * x3 - dt * (x3 * x3 * x3)
        plsc.store_scatter(outv, [xr, xc], o0)
        plsc.store_scatter(outv, [xr, xc + 1], o1)
        plsc.store_scatter(outv, [xr, xc + 2], o2)
        plsc.store_scatter(outv, [xr, xc + 3], o3)

    pltpu.sync_copy(outv, outf.at[pl.ds(w * ROW_W, ROW_W), :])


def kernel(X, graph_logits, mu_logits, alpha_param, beta_param, z, dt_val):
    Xf = X.reshape(XROWS, 128)
    zf = z.astype(jnp.int32)
    Gf = graph_logits.reshape(-1).astype(jnp.float32)
    muf = mu_logits.reshape(-1).astype(jnp.float32)
    dtf = jnp.full((L,), dt_val, dtype=jnp.float32)
    outf = _sc_forward(Xf, zf, Gf, muf,
                       alpha_param.astype(jnp.float32),
                       beta_param.astype(jnp.float32), dtf)
    return outf.reshape(B_TOK, N_NODES)
